# baseline (speedup 1.0000x reference)
import functools

import jax
import jax.numpy as jnp
from jax import lax
from jax.experimental import pallas as pl
from jax.experimental.pallas import tpu as pltpu
from jax.experimental.pallas import tpu_sc as plsc

_LEVELS = [10242, 2562, 642, 162, 42, 12]
_SPECS = [(128, 32, 0), (32, 32, 0), (32, 32, 0),
          (32, 64, 1), (64, 64, 1),
          (64, 128, 2), (128, 128, 2),
          (128, 256, 3), (256, 256, 3),
          (256, 512, 4), (512, 512, 4),
          (512, 1024, 5), (1024, 1024, 5)]
_NC, _NS = 2, 16
_EPS = 1e-5
_CP = 128


def _geom(n):
    ch = -(-n // (_NC * _NS))
    ch = -(-ch // 16) * 16
    wt = -(-n // ch)
    return ch, wt, ch * wt


def _chunks(ch):
    nck = -(-ch // 128)
    while ch % nck or (ch // nck) % 16:
        nck += 1
    s = ch // nck
    return [(i * s, s) for i in range(nck)]


def _mesh():
    return plsc.VectorSubcoreMesh(core_axis_name="c", subcore_axis_name="s",
                                  num_cores=_NC, num_subcores=_NS)


_SC_PARAMS = pltpu.CompilerParams(use_tc_tiling_on_sc=False,
                                  needs_layout_passes=False)


def _make_sc_conv(ch, wt, npad):
    s = 48 if ch % 48 == 0 else _chunks(ch)[0][1]
    nck = ch // s
    ncv = _CP // 16

    @functools.partial(
        pl.kernel,
        out_type=(jax.ShapeDtypeStruct((npad, _CP), jnp.float32),
                  jax.ShapeDtypeStruct((wt, 2, _CP), jnp.float32)),
        mesh=_mesh(),
        scratch_types=[
            pltpu.VMEM((7 * ch,), jnp.int32),
            pltpu.VMEM((2, 7, s), jnp.int32),
            pltpu.VMEM((2, 7, s, _CP), jnp.float32),
            pltpu.VMEM((s, _CP), jnp.float32),
            pltpu.VMEM((2, _CP), jnp.float32),
            pltpu.SemaphoreType.DMA,
            pltpu.SemaphoreType.DMA,
        ],
        compiler_params=_SC_PARAMS,
    )
    def k(yflat, noflat, out_hbm, stats_hbm, idxr, idxa, bufs, outb,
          statsv, sem0, sem1):
        wid = lax.axis_index("s") * _NC + lax.axis_index("c")
        sems = (sem0, sem1)

        @pl.when(wid < wt)
        def _():
            base = wid * ch
            pltpu.sync_copy(noflat.at[pl.ds(base * 7, ch * 7)], idxr)
            pat = lax.iota(jnp.int32, 16) * 7
            for jj in range(ncv):
                sl = pl.ds(jj * 16, 16)
                statsv[0, sl] = jnp.zeros((16,), jnp.float32)
                statsv[1, sl] = jnp.zeros((16,), jnp.float32)

            def fire(c):
                slot = c % 2
                for kk in range(7):
                    for j in range(s // 16):
                        v = plsc.load_gather(
                            idxr, [pat + ((c * s + j * 16) * 7 + kk)])
                        idxa[slot, kk, pl.ds(j * 16, 16)] = v + (kk * npad)
                return [pltpu.async_copy(yflat.at[idxa.at[slot, kk]],
                                         bufs.at[slot, kk], sems[slot])
                        for kk in range(7)]

            descs = fire(0)
            for c in range(nck):
                nxt = fire(c + 1) if c + 1 < nck else []
                for d in descs:
                    d.wait()
                descs = nxt
                slot = c % 2

                def body(i, carry):
                    for jj in range(ncv):
                        sl = pl.ds(jj * 16, 16)
                        v = bufs[slot, 0, i, sl]
                        for kk in range(1, 7):
                            v = v + bufs[slot, kk, i, sl]
                        outb[i, sl] = v
                        plsc.addupdate(statsv.at[0, sl], v)
                        plsc.addupdate(statsv.at[1, sl], v * v)
                    return carry

                lax.fori_loop(0, s, body, 0)
                pltpu.sync_copy(outb, out_hbm.at[pl.ds(base + c * s, s)])
            pltpu.sync_copy(statsv, stats_hbm.at[wid])

    return k


def _sc_rsqrt(v):
    i = plsc.bitcast(v, jnp.int32)
    y = plsc.bitcast(jnp.int32(0x5F3759DF) - (i >> 1), jnp.float32)
    for _ in range(4):
        y = y * (1.5 - 0.5 * v * y * y)
    return y


def _make_sc_pool(ch, wt, npad_c, wt_f, n_f):
    chunks = _chunks(ch)
    chunk = chunks[0][1]
    assert len(chunks) == 1 and chunk == ch
    ncv = _CP // 16

    @functools.partial(
        pl.kernel,
        out_type=jax.ShapeDtypeStruct((npad_c, _CP), jnp.float32),
        mesh=_mesh(),
        scratch_types=[
            pltpu.VMEM((7 * ch,), jnp.int32),
            pltpu.VMEM((7, ch), jnp.int32),
            pltpu.VMEM((7, ch, _CP), jnp.float32),
            pltpu.VMEM((wt_f, 2, _CP), jnp.float32),
            pltpu.VMEM((2, _CP), jnp.float32),
            pltpu.SemaphoreType.DMA,
        ],
        compiler_params=_SC_PARAMS,
    )
    def k(xf, noflat, stats, g2, be2, out_hbm, idxf, idxr, bufs, stv, ssv,
          sem):
        wid = lax.axis_index("s") * _NC + lax.axis_index("c")

        @pl.when(wid < wt)
        def _():
            base = wid * ch
            pltpu.sync_copy(noflat.at[pl.ds(base * 7, ch * 7)], idxf)
            pat = lax.iota(jnp.int32, 16) * 7
            for kk in range(7):
                for j in range(ch // 16):
                    idxr[kk, pl.ds(j * 16, 16)] = plsc.load_gather(
                        idxf, [pat + (j * 16 * 7 + kk)])
            pltpu.sync_copy(stats, stv)
            gdescs = [pltpu.async_copy(xf.at[idxr.at[kk]],
                                       bufs.at[kk], sem)
                      for kk in range(7)]
            pltpu.sync_copy(g2, ssv.at[pl.ds(0, 1)])
            pltpu.sync_copy(be2, ssv.at[pl.ds(1, 1)])
            for jj in range(ncv):
                sl = pl.ds(jj * 16, 16)
                s_v = stv[0, 0, sl]
                q_v = stv[0, 1, sl]
                for w in range(1, wt_f):
                    s_v = s_v + stv[w, 0, sl]
                    q_v = q_v + stv[w, 1, sl]
                m = s_v * (1.0 / n_f)
                var = q_v * (1.0 / n_f) - m * m
                r = _sc_rsqrt(var + _EPS)
                sc = ssv[0, sl] * r
                ssv[0, sl] = sc
                ssv[1, sl] = ssv[1, sl] - m * sc
            for d in gdescs:
                d.wait()

            def body(i, carry):
                for jj in range(ncv):
                    sl = pl.ds(jj * 16, 16)
                    sc = ssv[0, sl]
                    sh = ssv[1, sl]
                    acc = None
                    for kk in range(7):
                        a = bufs[kk, i, sl] * sc + sh
                        a = jnp.maximum(a, a * 0.1)
                        acc = a if acc is None else acc + a
                    bufs[0, i, sl] = acc * (1.0 / 7.0)
                return carry

            lax.fori_loop(0, ch, body, 0)
            pltpu.sync_copy(bufs.at[0], out_hbm.at[pl.ds(base, ch)])

    return k


def _finalize_stats(stats, g2, be2, n):
    sums = jnp.sum(stats, axis=0)
    m = sums[0] * (1.0 / n)
    v = sums[1] * (1.0 / n) - m * m
    scale = g2[0] * lax.rsqrt(v + _EPS)
    shift = be2[0] - m * scale
    return scale, shift


def _bn_act(x, g2, be2, n, npad, c):
    rid = lax.broadcasted_iota(jnp.int32, (npad, c), 0)
    msk = rid < n
    xm = jnp.where(msk, x, 0.0)
    s = jnp.sum(xm, axis=0)
    sq = jnp.sum(xm * xm, axis=0)
    m = s * (1.0 / n)
    v = sq * (1.0 / n) - m * m
    scale = g2[0] * lax.rsqrt(v + _EPS)
    shift = be2[0] - m * scale
    a = xm * scale[None, :] + shift[None, :]
    a = jnp.where(a >= 0, a, a * 0.1)
    return jnp.where(msk, a, 0.0)


def _make_tc_mm(npad, n, raw, wt=None):
    def prologue(x, stats, g2, be2):
        if raw:
            scale, shift = _finalize_stats(stats, g2, be2, n)
            a = x * scale[None, :] + shift[None, :]
            a = jnp.where(a >= 0, a, a * 0.1)
        else:
            a = x
        rid = lax.broadcasted_iota(jnp.int32, (npad, _CP), 0)
        return jnp.where(rid < n, a, 0.0)

    if raw:
        def body(x_ref, stats_ref, g_ref, be_ref, w_ref, o_ref, acc):
            @pl.when(pl.program_id(0) == 0)
            def _():
                acc[...] = prologue(x_ref[...], stats_ref[...],
                                    g_ref[...], be_ref[...])
            o_ref[...] = jnp.dot(acc[...], w_ref[0],
                                 preferred_element_type=jnp.float32)
        in_specs = [
            pl.BlockSpec((npad, _CP), lambda k: (0, 0)),
            pl.BlockSpec((wt, 2, _CP), lambda k: (0, 0, 0)),
            pl.BlockSpec((1, _CP), lambda k: (0, 0)),
            pl.BlockSpec((1, _CP), lambda k: (0, 0)),
            pl.BlockSpec((1, _CP, _CP), lambda k: (k, 0, 0)),
        ]
    else:
        def body(x_ref, w_ref, o_ref, acc):
            @pl.when(pl.program_id(0) == 0)
            def _():
                acc[...] = prologue(x_ref[...], None, None, None)
            o_ref[...] = jnp.dot(acc[...], w_ref[0],
                                 preferred_element_type=jnp.float32)
        in_specs = [
            pl.BlockSpec((npad, _CP), lambda k: (0, 0)),
            pl.BlockSpec((1, _CP, _CP), lambda k: (k, 0, 0)),
        ]

    return pl.pallas_call(
        body,
        grid=(7,),
        in_specs=in_specs,
        out_specs=pl.BlockSpec((npad, _CP), lambda k: (k, 0)),
        out_shape=jax.ShapeDtypeStruct((7 * npad, _CP), jnp.float32),
        scratch_shapes=[pltpu.VMEM((npad, _CP), jnp.float32)],
        compiler_params=pltpu.CompilerParams(
            dimension_semantics=("arbitrary",)),
    )


def _make_tc_finalize(wt, n):
    def body(stats_ref, g_ref, be_ref, o_ref):
        scale, shift = _finalize_stats(stats_ref[...], g_ref[...],
                                       be_ref[...], n)
        o_ref[...] = jnp.concatenate([scale[None, :], shift[None, :]],
                                     axis=0)

    return pl.pallas_call(
        body, out_shape=jax.ShapeDtypeStruct((2, _CP), jnp.float32))


def _onehot_pool(noTp, npad_c, npad_f):
    cols = lax.broadcasted_iota(jnp.int32, (npad_c, npad_f), 1)
    m = None
    for kk in range(7):
        p = (noTp[kk][:, None] == cols).astype(jnp.bfloat16)
        m = p if m is None else m + p
    return m


def _make_tc_mega(n_f, npad_f, c_f, n_c, npad_c, specs, last, raw_in=False):
    (ci1, co1), (ci2, co2) = specs

    def body(actf_ref, *refs):
        if raw_in:
            (stats_ref, gf_ref, bef_ref, noTp_ref, noT_ref, w1_ref,
             g1_ref, be1_ref, w2_ref, g2_ref, be2_ref, o_ref) = refs
            scale, shift = _finalize_stats(stats_ref[...], gf_ref[...],
                                           bef_ref[...], n_f)
            a = actf_ref[...] * scale[None, :] + shift[None, :]
            a = jnp.where(a >= 0, a, a * 0.1)
            rid = lax.broadcasted_iota(jnp.int32, (npad_f, a.shape[1]), 0)
            actf = jnp.where(rid < n_f, a, 0.0)
        else:
            (noTp_ref, noT_ref, w1_ref, g1_ref, be1_ref, w2_ref, g2_ref,
             be2_ref, o_ref) = refs
            actf = actf_ref[...]
        M = _onehot_pool(noTp_ref[...], npad_c, npad_f)
        x = jnp.dot(M, actf.astype(jnp.bfloat16),
                    preferred_element_type=jnp.float32) * (1.0 / 7.0)
        x = x[:, :ci1]
        cols = lax.broadcasted_iota(jnp.int32, (npad_c, npad_c), 1)
        noT = noT_ref[...]
        Pk = [(noT[kk][:, None] == cols).astype(jnp.bfloat16)
              for kk in range(7)]
        for (ci, co, w_ref, g_ref, be_ref) in (
                (ci1, co1, w1_ref, g1_ref, be1_ref),
                (ci2, co2, w2_ref, g2_ref, be2_ref)):
            xb = x.astype(jnp.bfloat16)
            mat = jnp.concatenate(
                [jnp.dot(p, xb, preferred_element_type=jnp.float32)
                 for p in Pk], axis=1)
            w = w_ref[...]
            if w.dtype == jnp.bfloat16:
                mat = mat.astype(jnp.bfloat16)
            out = jnp.dot(mat, w, preferred_element_type=jnp.float32)
            x = _bn_act(out, g_ref[...], be_ref[...], n_c, npad_c, co)
        if last:
            o_ref[...] = jnp.sum(x, axis=0, keepdims=True) * (1.0 / n_c)
        else:
            o_ref[...] = x

    out_shape = jax.ShapeDtypeStruct(
        (1, co2) if last else (npad_c, co2), jnp.float32)
    return pl.pallas_call(body, out_shape=out_shape)


def _pad2(a, rows, cols):
    return jnp.pad(a, ((0, rows - a.shape[0]), (0, cols - a.shape[1])))


def kernel(x, no0, no1, no2, no3, no4, no5,
           W0, b0, g0, be0, W1, b1, g1, be1, W2, b2, g2, be2,
           W3, b3, g3, be3, W4, b4, g4, be4, W5, b5, g5, be5,
           W6, b6, g6, be6, W7, b7, g7, be7, W8, b8, g8, be8,
           W9, b9, g9, be9, W10, b10, g10, be10, W11, b11, g11, be11,
           W12, b12, g12, be12):
    nos = [no0, no1, no2, no3, no4, no5]
    Ws = [W0, W1, W2, W3, W4, W5, W6, W7, W8, W9, W10, W11, W12]
    gs = [g0, g1, g2, g3, g4, g5, g6, g7, g8, g9, g10, g11, g12]
    bes = [be0, be1, be2, be3, be4, be5, be6, be7, be8, be9, be10, be11,
           be12]

    geoms = [_geom(n) for n in _LEVELS[:2]]

    sig32 = [2 * p for p in range(16)] + [2 * p + 1 for p in range(16)]
    sig = jnp.asarray([g * 32 + p for g in range(_CP // 32) for p in sig32],
                      dtype=jnp.int32)

    noflats = []
    for l in range(2):
        n = _LEVELS[l]
        _, _, npad = geoms[l]
        noflats.append(jnp.pad(nos[l], (0, 7 * (npad - n)),
                               constant_values=n))
    n_c1 = _LEVELS[1]
    ch1, wt1, npad1 = geoms[1]
    noflat_p1 = jnp.pad(nos[0][:n_c1 * 7], (0, 7 * (npad1 - n_c1)))

    W3d = []
    g2s = []
    be2s = []
    for j, (ci, co, lv) in enumerate(_SPECS):
        if j <= 4:
            w = Ws[j].reshape(7, ci, co)
            w = jnp.pad(w, ((0, 0), (0, _CP - ci), (0, _CP - co)))
            W3d.append(w)
        else:
            W3d.append(Ws[j])
        c = max(co, _CP)
        g2 = jnp.pad(gs[j], (0, c - co)).reshape(1, c)
        be2 = jnp.pad(bes[j], (0, c - co)).reshape(1, c)
        g2s.append(g2)
        be2s.append(be2)

    act = x
    state_raw = None
    out = stats = None
    for j in range(5):
        ci, co, lv = _SPECS[j]
        n = _LEVELS[lv]
        ch, wt, npad = geoms[lv]
        if state_raw is None:
            Y = _make_tc_mm(npad, n, raw=False)(act, W3d[j])
        else:
            out_p, stats_p = state_raw
            Y = _make_tc_mm(npad, n, raw=True, wt=wt)(
                out_p, stats_p, g2s[j - 1], be2s[j - 1], W3d[j])
        out, stats = _make_sc_conv(ch, wt, npad)(Y, noflats[lv])
        state_raw = (out, stats)

        if j == 2:
            _, wt0, _ = geoms[0]
            act = _make_sc_pool(ch1, wt1, npad1, wt0, _LEVELS[0])(
                out, noflat_p1, stats, g2s[2], be2s[2])
            state_raw = None

    npads = {1: npad1, 2: 672, 3: 168, 4: 48, 5: 16}
    for l in (2, 3, 4, 5):
        n_f, n_c = _LEVELS[l - 1], _LEVELS[l]
        npad_f, npad_c = npads[l - 1], npads[l]
        j1, j2 = 2 * l + 1, 2 * l + 2
        ci1, co1, _ = _SPECS[j1]
        ci2, co2, _ = _SPECS[j2]
        tp = nos[l - 1][:n_c * 7].reshape(n_c, 7).T
        tp = jnp.pad(tp, ((0, 0), (0, npad_c - n_c)), constant_values=0)
        t = nos[l].reshape(n_c, 7).T
        t = jnp.pad(t, ((0, 0), (0, npad_c - n_c)), constant_values=0)
        w1m, w2m = Ws[j1], Ws[j2]
        if l == 5:
            w1m = w1m.astype(jnp.bfloat16)
            w2m = w2m.astype(jnp.bfloat16)
        mega = _make_tc_mega(n_f, npad_f, max(ci1, _CP), n_c, npad_c,
                             [(ci1, co1), (ci2, co2)], last=(l == 5),
                             raw_in=(l == 2))
        if l == 2:
            act = mega(out, stats, g2s[4], be2s[4], tp, t,
                       w1m, g2s[j1], be2s[j1],
                       w2m, g2s[j2], be2s[j2])
        else:
            act = mega(act, tp, t, w1m, g2s[j1], be2s[j1],
                       w2m, g2s[j2], be2s[j2])
    return act


# speedup vs baseline: 1.0092x; 1.0092x over previous
import functools

import jax
import jax.numpy as jnp
from jax import lax
from jax.experimental import pallas as pl
from jax.experimental.pallas import tpu as pltpu
from jax.experimental.pallas import tpu_sc as plsc

_LEVELS = [10242, 2562, 642, 162, 42, 12]
_SPECS = [(128, 32, 0), (32, 32, 0), (32, 32, 0),
          (32, 64, 1), (64, 64, 1),
          (64, 128, 2), (128, 128, 2),
          (128, 256, 3), (256, 256, 3),
          (256, 512, 4), (512, 512, 4),
          (512, 1024, 5), (1024, 1024, 5)]
_NC, _NS = 2, 16
_EPS = 1e-5
_CP = 128


def _geom(n):
    ch = -(-n // (_NC * _NS))
    ch = -(-ch // 16) * 16
    wt = -(-n // ch)
    return ch, wt, ch * wt


def _chunks(ch):
    nck = -(-ch // 128)
    while ch % nck or (ch // nck) % 16:
        nck += 1
    s = ch // nck
    return [(i * s, s) for i in range(nck)]


def _mesh():
    return plsc.VectorSubcoreMesh(core_axis_name="c", subcore_axis_name="s",
                                  num_cores=_NC, num_subcores=_NS)


_SC_PARAMS = pltpu.CompilerParams(use_tc_tiling_on_sc=False,
                                  needs_layout_passes=False)


def _make_sc_conv(ch, wt, npad):
    s = 48 if ch % 48 == 0 else _chunks(ch)[0][1]
    nck = ch // s
    ncv = _CP // 16

    @functools.partial(
        pl.kernel,
        out_type=(jax.ShapeDtypeStruct((npad, _CP), jnp.float32),
                  jax.ShapeDtypeStruct((wt, 2, _CP), jnp.float32)),
        mesh=_mesh(),
        scratch_types=[
            pltpu.VMEM((7 * ch,), jnp.int32),
            pltpu.VMEM((2, 7, s), jnp.int32),
            pltpu.VMEM((2, 7, s, _CP), jnp.float32),
            pltpu.VMEM((s, _CP), jnp.float32),
            pltpu.VMEM((2, _CP), jnp.float32),
            pltpu.SemaphoreType.DMA,
            pltpu.SemaphoreType.DMA,
        ],
        compiler_params=_SC_PARAMS,
    )
    def k(yflat, noflat, out_hbm, stats_hbm, idxr, idxa, bufs, outb,
          statsv, sem0, sem1):
        wid = lax.axis_index("s") * _NC + lax.axis_index("c")
        sems = (sem0, sem1)

        @pl.when(wid < wt)
        def _():
            base = wid * ch
            pltpu.sync_copy(noflat.at[pl.ds(base * 7, ch * 7)], idxr)
            pat = lax.iota(jnp.int32, 16) * 7
            for jj in range(ncv):
                sl = pl.ds(jj * 16, 16)
                statsv[0, sl] = jnp.zeros((16,), jnp.float32)
                statsv[1, sl] = jnp.zeros((16,), jnp.float32)

            def fire(c):
                slot = c % 2
                for kk in range(7):
                    for j in range(s // 16):
                        v = plsc.load_gather(
                            idxr, [pat + ((c * s + j * 16) * 7 + kk)])
                        idxa[slot, kk, pl.ds(j * 16, 16)] = v + (kk * npad)
                return [pltpu.async_copy(yflat.at[idxa.at[slot, kk]],
                                         bufs.at[slot, kk], sems[slot])
                        for kk in range(7)]

            descs = fire(0)
            for c in range(nck):
                nxt = fire(c + 1) if c + 1 < nck else []
                for d in descs:
                    d.wait()
                descs = nxt
                slot = c % 2

                def body(i, carry):
                    for jj in range(ncv):
                        sl = pl.ds(jj * 16, 16)
                        v = bufs[slot, 0, i, sl]
                        for kk in range(1, 7):
                            v = v + bufs[slot, kk, i, sl]
                        outb[i, sl] = v
                        plsc.addupdate(statsv.at[0, sl], v)
                        plsc.addupdate(statsv.at[1, sl], v * v)
                    return carry

                lax.fori_loop(0, s, body, 0)
                pltpu.sync_copy(outb, out_hbm.at[pl.ds(base + c * s, s)])
            pltpu.sync_copy(statsv, stats_hbm.at[wid])

    return k


def _sc_rsqrt(v):
    i = plsc.bitcast(v, jnp.int32)
    y = plsc.bitcast(jnp.int32(0x5F3759DF) - (i >> 1), jnp.float32)
    for _ in range(4):
        y = y * (1.5 - 0.5 * v * y * y)
    return y


def _make_sc_pool(ch, wt, npad_c, wt_f, n_f):
    s = 48 if ch % 48 == 0 else ch
    nck = ch // s
    ncv = _CP // 16

    @functools.partial(
        pl.kernel,
        out_type=jax.ShapeDtypeStruct((npad_c, _CP), jnp.float32),
        mesh=_mesh(),
        scratch_types=[
            pltpu.VMEM((7 * ch,), jnp.int32),
            pltpu.VMEM((2, 7, s), jnp.int32),
            pltpu.VMEM((2, 7, s, _CP), jnp.float32),
            pltpu.VMEM((s, _CP), jnp.float32),
            pltpu.VMEM((wt_f, 2, _CP), jnp.float32),
            pltpu.VMEM((2, _CP), jnp.float32),
            pltpu.SemaphoreType.DMA,
            pltpu.SemaphoreType.DMA,
        ],
        compiler_params=_SC_PARAMS,
    )
    def k(xf, noflat, stats, g2, be2, out_hbm, idxf, idxr, bufs, outb,
          stv, ssv, sem0, sem1):
        wid = lax.axis_index("s") * _NC + lax.axis_index("c")
        sems = (sem0, sem1)

        @pl.when(wid < wt)
        def _():
            base = wid * ch
            pltpu.sync_copy(noflat.at[pl.ds(base * 7, ch * 7)], idxf)
            pat = lax.iota(jnp.int32, 16) * 7

            def fire(c):
                slot = c % 2
                for kk in range(7):
                    for j in range(s // 16):
                        idxr[slot, kk, pl.ds(j * 16, 16)] = \
                            plsc.load_gather(
                                idxf, [pat + ((c * s + j * 16) * 7 + kk)])
                return [pltpu.async_copy(xf.at[idxr.at[slot, kk]],
                                         bufs.at[slot, kk], sems[slot])
                        for kk in range(7)]

            descs = fire(0)
            pltpu.sync_copy(stats, stv)
            pltpu.sync_copy(g2, ssv.at[pl.ds(0, 1)])
            pltpu.sync_copy(be2, ssv.at[pl.ds(1, 1)])
            for jj in range(ncv):
                sl = pl.ds(jj * 16, 16)
                s_v = stv[0, 0, sl]
                q_v = stv[0, 1, sl]
                for w in range(1, wt_f):
                    s_v = s_v + stv[w, 0, sl]
                    q_v = q_v + stv[w, 1, sl]
                m = s_v * (1.0 / n_f)
                var = q_v * (1.0 / n_f) - m * m
                r = _sc_rsqrt(var + _EPS)
                sc = ssv[0, sl] * r
                ssv[0, sl] = sc
                ssv[1, sl] = ssv[1, sl] - m * sc
            for c in range(nck):
                nxt = fire(c + 1) if c + 1 < nck else []
                for d in descs:
                    d.wait()
                descs = nxt
                slot = c % 2

                def body(i, carry):
                    for jj in range(ncv):
                        sl = pl.ds(jj * 16, 16)
                        sc = ssv[0, sl]
                        sh = ssv[1, sl]
                        acc = None
                        for kk in range(7):
                            a = bufs[slot, kk, i, sl] * sc + sh
                            a = jnp.maximum(a, a * 0.1)
                            acc = a if acc is None else acc + a
                        outb[i, sl] = acc * (1.0 / 7.0)
                    return carry

                lax.fori_loop(0, s, body, 0)
                pltpu.sync_copy(outb, out_hbm.at[pl.ds(base + c * s, s)])

    return k


def _finalize_stats(stats, g2, be2, n):
    sums = jnp.sum(stats, axis=0)
    m = sums[0] * (1.0 / n)
    v = sums[1] * (1.0 / n) - m * m
    scale = g2[0] * lax.rsqrt(v + _EPS)
    shift = be2[0] - m * scale
    return scale, shift


def _bn_act(x, g2, be2, n, npad, c):
    rid = lax.broadcasted_iota(jnp.int32, (npad, c), 0)
    msk = rid < n
    xm = jnp.where(msk, x, 0.0)
    s = jnp.sum(xm, axis=0)
    sq = jnp.sum(xm * xm, axis=0)
    m = s * (1.0 / n)
    v = sq * (1.0 / n) - m * m
    scale = g2[0] * lax.rsqrt(v + _EPS)
    shift = be2[0] - m * scale
    a = xm * scale[None, :] + shift[None, :]
    a = jnp.where(a >= 0, a, a * 0.1)
    return jnp.where(msk, a, 0.0)


def _make_tc_mm(npad, n, raw, wt=None):
    def prologue(x, stats, g2, be2):
        if raw:
            scale, shift = _finalize_stats(stats, g2, be2, n)
            a = x * scale[None, :] + shift[None, :]
            a = jnp.where(a >= 0, a, a * 0.1)
        else:
            a = x
        rid = lax.broadcasted_iota(jnp.int32, (npad, _CP), 0)
        return jnp.where(rid < n, a, 0.0)

    if raw:
        def body(x_ref, stats_ref, g_ref, be_ref, w_ref, o_ref, acc):
            @pl.when(pl.program_id(0) == 0)
            def _():
                acc[...] = prologue(x_ref[...], stats_ref[...],
                                    g_ref[...], be_ref[...])
            o_ref[...] = jnp.dot(acc[...], w_ref[0],
                                 preferred_element_type=jnp.float32)
        in_specs = [
            pl.BlockSpec((npad, _CP), lambda k: (0, 0)),
            pl.BlockSpec((wt, 2, _CP), lambda k: (0, 0, 0)),
            pl.BlockSpec((1, _CP), lambda k: (0, 0)),
            pl.BlockSpec((1, _CP), lambda k: (0, 0)),
            pl.BlockSpec((1, _CP, _CP), lambda k: (k, 0, 0)),
        ]
    else:
        def body(x_ref, w_ref, o_ref, acc):
            @pl.when(pl.program_id(0) == 0)
            def _():
                acc[...] = prologue(x_ref[...], None, None, None)
            o_ref[...] = jnp.dot(acc[...], w_ref[0],
                                 preferred_element_type=jnp.float32)
        in_specs = [
            pl.BlockSpec((npad, _CP), lambda k: (0, 0)),
            pl.BlockSpec((1, _CP, _CP), lambda k: (k, 0, 0)),
        ]

    return pl.pallas_call(
        body,
        grid=(7,),
        in_specs=in_specs,
        out_specs=pl.BlockSpec((npad, _CP), lambda k: (k, 0)),
        out_shape=jax.ShapeDtypeStruct((7 * npad, _CP), jnp.float32),
        scratch_shapes=[pltpu.VMEM((npad, _CP), jnp.float32)],
        compiler_params=pltpu.CompilerParams(
            dimension_semantics=("arbitrary",)),
    )


def _make_tc_finalize(wt, n):
    def body(stats_ref, g_ref, be_ref, o_ref):
        scale, shift = _finalize_stats(stats_ref[...], g_ref[...],
                                       be_ref[...], n)
        o_ref[...] = jnp.concatenate([scale[None, :], shift[None, :]],
                                     axis=0)

    return pl.pallas_call(
        body, out_shape=jax.ShapeDtypeStruct((2, _CP), jnp.float32))


def _onehot_pool(noTp, npad_c, npad_f):
    cols = lax.broadcasted_iota(jnp.int32, (npad_c, npad_f), 1)
    m = None
    for kk in range(7):
        p = (noTp[kk][:, None] == cols).astype(jnp.float32)
        m = p if m is None else m + p
    return m * (1.0 / 7.0)


def _make_tc_mega(n_f, npad_f, c_f, n_c, npad_c, specs, last, raw_in=False):
    (ci1, co1), (ci2, co2) = specs

    def body(actf_ref, *refs):
        if raw_in:
            (stats_ref, gf_ref, bef_ref, noTp_ref, noT_ref, w1_ref,
             g1_ref, be1_ref, w2_ref, g2_ref, be2_ref, o_ref) = refs
            scale, shift = _finalize_stats(stats_ref[...], gf_ref[...],
                                           bef_ref[...], n_f)
            a = actf_ref[...] * scale[None, :] + shift[None, :]
            a = jnp.where(a >= 0, a, a * 0.1)
            rid = lax.broadcasted_iota(jnp.int32, (npad_f, a.shape[1]), 0)
            actf = jnp.where(rid < n_f, a, 0.0)
        else:
            (noTp_ref, noT_ref, w1_ref, g1_ref, be1_ref, w2_ref, g2_ref,
             be2_ref, o_ref) = refs
            actf = actf_ref[...]
        M = _onehot_pool(noTp_ref[...], npad_c, npad_f)
        x = jnp.dot(M, actf, preferred_element_type=jnp.float32)
        x = x[:, :ci1]
        cols = lax.broadcasted_iota(jnp.int32, (npad_c, npad_c), 1)
        noT = noT_ref[...]
        Pk = [(noT[kk][:, None] == cols).astype(jnp.float32)
              for kk in range(7)]
        for (ci, co, w_ref, g_ref, be_ref) in (
                (ci1, co1, w1_ref, g1_ref, be1_ref),
                (ci2, co2, w2_ref, g2_ref, be2_ref)):
            mat = jnp.concatenate(
                [jnp.dot(p, x, preferred_element_type=jnp.float32)
                 for p in Pk], axis=1)
            w = w_ref[...]
            if w.dtype == jnp.bfloat16:
                mat = mat.astype(jnp.bfloat16)
            out = jnp.dot(mat, w, preferred_element_type=jnp.float32)
            x = _bn_act(out, g_ref[...], be_ref[...], n_c, npad_c, co)
        if last:
            o_ref[...] = jnp.sum(x, axis=0, keepdims=True) * (1.0 / n_c)
        else:
            o_ref[...] = x

    out_shape = jax.ShapeDtypeStruct(
        (1, co2) if last else (npad_c, co2), jnp.float32)
    return pl.pallas_call(body, out_shape=out_shape)


def _pad2(a, rows, cols):
    return jnp.pad(a, ((0, rows - a.shape[0]), (0, cols - a.shape[1])))


def kernel(x, no0, no1, no2, no3, no4, no5,
           W0, b0, g0, be0, W1, b1, g1, be1, W2, b2, g2, be2,
           W3, b3, g3, be3, W4, b4, g4, be4, W5, b5, g5, be5,
           W6, b6, g6, be6, W7, b7, g7, be7, W8, b8, g8, be8,
           W9, b9, g9, be9, W10, b10, g10, be10, W11, b11, g11, be11,
           W12, b12, g12, be12):
    nos = [no0, no1, no2, no3, no4, no5]
    Ws = [W0, W1, W2, W3, W4, W5, W6, W7, W8, W9, W10, W11, W12]
    gs = [g0, g1, g2, g3, g4, g5, g6, g7, g8, g9, g10, g11, g12]
    bes = [be0, be1, be2, be3, be4, be5, be6, be7, be8, be9, be10, be11,
           be12]

    geoms = [_geom(n) for n in _LEVELS[:2]]

    sig32 = [2 * p for p in range(16)] + [2 * p + 1 for p in range(16)]
    sig = jnp.asarray([g * 32 + p for g in range(_CP // 32) for p in sig32],
                      dtype=jnp.int32)

    noflats = []
    for l in range(2):
        n = _LEVELS[l]
        _, _, npad = geoms[l]
        noflats.append(jnp.pad(nos[l], (0, 7 * (npad - n)),
                               constant_values=n))
    n_c1 = _LEVELS[1]
    ch1, wt1, npad1 = geoms[1]
    noflat_p1 = jnp.pad(nos[0][:n_c1 * 7], (0, 7 * (npad1 - n_c1)))

    W3d = []
    g2s = []
    be2s = []
    for j, (ci, co, lv) in enumerate(_SPECS):
        if j <= 4:
            w = Ws[j].reshape(7, ci, co)
            w = jnp.pad(w, ((0, 0), (0, _CP - ci), (0, _CP - co)))
            W3d.append(w)
        else:
            W3d.append(Ws[j])
        c = max(co, _CP)
        g2 = jnp.pad(gs[j], (0, c - co)).reshape(1, c)
        be2 = jnp.pad(bes[j], (0, c - co)).reshape(1, c)
        g2s.append(g2)
        be2s.append(be2)

    act = x
    state_raw = None
    out = stats = None
    for j in range(5):
        ci, co, lv = _SPECS[j]
        n = _LEVELS[lv]
        ch, wt, npad = geoms[lv]
        if state_raw is None:
            Y = _make_tc_mm(npad, n, raw=False)(act, W3d[j])
        else:
            out_p, stats_p = state_raw
            Y = _make_tc_mm(npad, n, raw=True, wt=wt)(
                out_p, stats_p, g2s[j - 1], be2s[j - 1], W3d[j])
        out, stats = _make_sc_conv(ch, wt, npad)(Y, noflats[lv])
        state_raw = (out, stats)

        if j == 2:
            _, wt0, _ = geoms[0]
            act = _make_sc_pool(ch1, wt1, npad1, wt0, _LEVELS[0])(
                out, noflat_p1, stats, g2s[2], be2s[2])
            state_raw = None

    npads = {1: npad1, 2: 672, 3: 168, 4: 48, 5: 16}
    for l in (2, 3, 4, 5):
        n_f, n_c = _LEVELS[l - 1], _LEVELS[l]
        npad_f, npad_c = npads[l - 1], npads[l]
        j1, j2 = 2 * l + 1, 2 * l + 2
        ci1, co1, _ = _SPECS[j1]
        ci2, co2, _ = _SPECS[j2]
        tp = nos[l - 1][:n_c * 7].reshape(n_c, 7).T
        tp = jnp.pad(tp, ((0, 0), (0, npad_c - n_c)), constant_values=0)
        t = nos[l].reshape(n_c, 7).T
        t = jnp.pad(t, ((0, 0), (0, npad_c - n_c)), constant_values=0)
        w1m, w2m = Ws[j1], Ws[j2]
        if l == 5:
            w1m = w1m.astype(jnp.bfloat16)
            w2m = w2m.astype(jnp.bfloat16)
        mega = _make_tc_mega(n_f, npad_f, max(ci1, _CP), n_c, npad_c,
                             [(ci1, co1), (ci2, co2)], last=(l == 5),
                             raw_in=(l == 2))
        if l == 2:
            act = mega(out, stats, g2s[4], be2s[4], tp, t,
                       w1m, g2s[j1], be2s[j1],
                       w2m, g2s[j2], be2s[j2])
        else:
            act = mega(act, tp, t, w1m, g2s[j1], be2s[j1],
                       w2m, g2s[j2], be2s[j2])
    return act


# speedup vs baseline: 1.0405x; 1.0311x over previous
import functools

import jax
import jax.numpy as jnp
from jax import lax
from jax.experimental import pallas as pl
from jax.experimental.pallas import tpu as pltpu
from jax.experimental.pallas import tpu_sc as plsc

_LEVELS = [10242, 2562, 642, 162, 42, 12]
_SPECS = [(128, 32, 0), (32, 32, 0), (32, 32, 0),
          (32, 64, 1), (64, 64, 1),
          (64, 128, 2), (128, 128, 2),
          (128, 256, 3), (256, 256, 3),
          (256, 512, 4), (512, 512, 4),
          (512, 1024, 5), (1024, 1024, 5)]
_NC, _NS = 2, 16
_EPS = 1e-5
_CP = 128


def _geom(n):
    ch = -(-n // (_NC * _NS))
    ch = -(-ch // 16) * 16
    wt = -(-n // ch)
    return ch, wt, ch * wt


def _chunks(ch):
    nck = -(-ch // 128)
    while ch % nck or (ch // nck) % 16:
        nck += 1
    s = ch // nck
    return [(i * s, s) for i in range(nck)]


def _mesh():
    return plsc.VectorSubcoreMesh(core_axis_name="c", subcore_axis_name="s",
                                  num_cores=_NC, num_subcores=_NS)


_SC_PARAMS = pltpu.CompilerParams(use_tc_tiling_on_sc=False,
                                  needs_layout_passes=False)


def _make_sc_conv(ch, wt, npad):
    s = 48 if ch % 48 == 0 else _chunks(ch)[0][1]
    nck = ch // s
    ncv = _CP // 16

    @functools.partial(
        pl.kernel,
        out_type=(jax.ShapeDtypeStruct((npad, _CP), jnp.float32),
                  jax.ShapeDtypeStruct((wt, 2, _CP), jnp.float32)),
        mesh=_mesh(),
        scratch_types=[
            pltpu.VMEM((7 * ch,), jnp.int32),
            pltpu.VMEM((2, 7, s), jnp.int32),
            pltpu.VMEM((2, 7, s, _CP), jnp.float32),
            pltpu.VMEM((s, _CP), jnp.float32),
            pltpu.VMEM((2, _CP), jnp.float32),
            pltpu.SemaphoreType.DMA,
            pltpu.SemaphoreType.DMA,
        ],
        compiler_params=_SC_PARAMS,
    )
    def k(yflat, noflat, out_hbm, stats_hbm, idxr, idxa, bufs, outb,
          statsv, sem0, sem1):
        wid = lax.axis_index("s") * _NC + lax.axis_index("c")
        sems = (sem0, sem1)

        @pl.when(wid < wt)
        def _():
            base = wid * ch
            pltpu.sync_copy(noflat.at[pl.ds(base * 7, ch * 7)], idxr)
            pat = lax.iota(jnp.int32, 16) * 7
            for jj in range(ncv):
                sl = pl.ds(jj * 16, 16)
                statsv[0, sl] = jnp.zeros((16,), jnp.float32)
                statsv[1, sl] = jnp.zeros((16,), jnp.float32)

            def fire(c):
                slot = c % 2
                for kk in range(7):
                    for j in range(s // 16):
                        v = plsc.load_gather(
                            idxr, [pat + ((c * s + j * 16) * 7 + kk)])
                        idxa[slot, kk, pl.ds(j * 16, 16)] = v + (kk * npad)
                return [pltpu.async_copy(yflat.at[idxa.at[slot, kk]],
                                         bufs.at[slot, kk], sems[slot])
                        for kk in range(7)]

            descs = fire(0)
            for c in range(nck):
                nxt = fire(c + 1) if c + 1 < nck else []
                for d in descs:
                    d.wait()
                descs = nxt
                slot = c % 2

                def body(i, carry):
                    for jj in range(ncv):
                        sl = pl.ds(jj * 16, 16)
                        v = bufs[slot, 0, i, sl]
                        for kk in range(1, 7):
                            v = v + bufs[slot, kk, i, sl]
                        outb[i, sl] = v
                        plsc.addupdate(statsv.at[0, sl], v)
                        plsc.addupdate(statsv.at[1, sl], v * v)
                    return carry

                lax.fori_loop(0, s, body, 0)
                pltpu.sync_copy(outb, out_hbm.at[pl.ds(base + c * s, s)])
            pltpu.sync_copy(statsv, stats_hbm.at[wid])

    return k


def _sc_rsqrt(v):
    i = plsc.bitcast(v, jnp.int32)
    y = plsc.bitcast(jnp.int32(0x5F3759DF) - (i >> 1), jnp.float32)
    for _ in range(4):
        y = y * (1.5 - 0.5 * v * y * y)
    return y


def _make_sc_pool(ch, wt, npad_c, wt_f, n_f):
    s = 48 if ch % 48 == 0 else ch
    nck = ch // s
    ncv = _CP // 16

    @functools.partial(
        pl.kernel,
        out_type=jax.ShapeDtypeStruct((npad_c, _CP), jnp.float32),
        mesh=_mesh(),
        scratch_types=[
            pltpu.VMEM((7 * ch,), jnp.int32),
            pltpu.VMEM((2, 7, s), jnp.int32),
            pltpu.VMEM((2, 7, s, _CP), jnp.float32),
            pltpu.VMEM((s, _CP), jnp.float32),
            pltpu.VMEM((wt_f, 2, _CP), jnp.float32),
            pltpu.VMEM((2, _CP), jnp.float32),
            pltpu.SemaphoreType.DMA,
            pltpu.SemaphoreType.DMA,
        ],
        compiler_params=_SC_PARAMS,
    )
    def k(xf, noflat, stats, g2, be2, out_hbm, idxf, idxr, bufs, outb,
          stv, ssv, sem0, sem1):
        wid = lax.axis_index("s") * _NC + lax.axis_index("c")
        sems = (sem0, sem1)

        @pl.when(wid < wt)
        def _():
            base = wid * ch
            pltpu.sync_copy(noflat.at[pl.ds(base * 7, ch * 7)], idxf)
            pat = lax.iota(jnp.int32, 16) * 7

            def fire(c):
                slot = c % 2
                for kk in range(7):
                    for j in range(s // 16):
                        idxr[slot, kk, pl.ds(j * 16, 16)] = \
                            plsc.load_gather(
                                idxf, [pat + ((c * s + j * 16) * 7 + kk)])
                return [pltpu.async_copy(xf.at[idxr.at[slot, kk]],
                                         bufs.at[slot, kk], sems[slot])
                        for kk in range(7)]

            descs = fire(0)
            pltpu.sync_copy(stats, stv)
            pltpu.sync_copy(g2, ssv.at[pl.ds(0, 1)])
            pltpu.sync_copy(be2, ssv.at[pl.ds(1, 1)])
            for jj in range(ncv):
                sl = pl.ds(jj * 16, 16)
                s_v = stv[0, 0, sl]
                q_v = stv[0, 1, sl]
                for w in range(1, wt_f):
                    s_v = s_v + stv[w, 0, sl]
                    q_v = q_v + stv[w, 1, sl]
                m = s_v * (1.0 / n_f)
                var = q_v * (1.0 / n_f) - m * m
                r = _sc_rsqrt(var + _EPS)
                sc = ssv[0, sl] * r
                ssv[0, sl] = sc
                ssv[1, sl] = ssv[1, sl] - m * sc
            for c in range(nck):
                nxt = fire(c + 1) if c + 1 < nck else []
                for d in descs:
                    d.wait()
                descs = nxt
                slot = c % 2

                def body(i, carry):
                    for jj in range(ncv):
                        sl = pl.ds(jj * 16, 16)
                        sc = ssv[0, sl]
                        sh = ssv[1, sl]
                        acc = None
                        for kk in range(7):
                            a = bufs[slot, kk, i, sl] * sc + sh
                            a = jnp.maximum(a, a * 0.1)
                            acc = a if acc is None else acc + a
                        outb[i, sl] = acc * (1.0 / 7.0)
                    return carry

                lax.fori_loop(0, s, body, 0)
                pltpu.sync_copy(outb, out_hbm.at[pl.ds(base + c * s, s)])

    return k


def _finalize_stats(stats, g2, be2, n):
    sums = jnp.sum(stats, axis=0)
    m = sums[0] * (1.0 / n)
    v = sums[1] * (1.0 / n) - m * m
    scale = g2[0] * lax.rsqrt(v + _EPS)
    shift = be2[0] - m * scale
    return scale, shift


def _bn_act(x, g2, be2, n, npad, c):
    rid = lax.broadcasted_iota(jnp.int32, (npad, c), 0)
    msk = rid < n
    xm = jnp.where(msk, x, 0.0)
    s = jnp.sum(xm, axis=0)
    sq = jnp.sum(xm * xm, axis=0)
    m = s * (1.0 / n)
    v = sq * (1.0 / n) - m * m
    scale = g2[0] * lax.rsqrt(v + _EPS)
    shift = be2[0] - m * scale
    a = xm * scale[None, :] + shift[None, :]
    a = jnp.where(a >= 0, a, a * 0.1)
    return jnp.where(msk, a, 0.0)


def _make_tc_mm(npad, n, raw, wt=None):
    def prologue(x, stats, g2, be2):
        if raw:
            scale, shift = _finalize_stats(stats, g2, be2, n)
            a = x * scale[None, :] + shift[None, :]
            a = jnp.where(a >= 0, a, a * 0.1)
        else:
            a = x
        rid = lax.broadcasted_iota(jnp.int32, (npad, _CP), 0)
        return jnp.where(rid < n, a, 0.0)

    if raw:
        def body(x_ref, stats_ref, g_ref, be_ref, w_ref, o_ref, acc):
            @pl.when(pl.program_id(0) == 0)
            def _():
                acc[...] = prologue(x_ref[...], stats_ref[...],
                                    g_ref[...], be_ref[...])
            o_ref[...] = jnp.dot(acc[...], w_ref[0],
                                 preferred_element_type=jnp.float32)
        in_specs = [
            pl.BlockSpec((npad, _CP), lambda k: (0, 0)),
            pl.BlockSpec((wt, 2, _CP), lambda k: (0, 0, 0)),
            pl.BlockSpec((1, _CP), lambda k: (0, 0)),
            pl.BlockSpec((1, _CP), lambda k: (0, 0)),
            pl.BlockSpec((1, _CP, _CP), lambda k: (k, 0, 0)),
        ]
    else:
        def body(x_ref, w_ref, o_ref, acc):
            @pl.when(pl.program_id(0) == 0)
            def _():
                acc[...] = prologue(x_ref[...], None, None, None)
            o_ref[...] = jnp.dot(acc[...], w_ref[0],
                                 preferred_element_type=jnp.float32)
        in_specs = [
            pl.BlockSpec((npad, _CP), lambda k: (0, 0)),
            pl.BlockSpec((1, _CP, _CP), lambda k: (k, 0, 0)),
        ]

    return pl.pallas_call(
        body,
        grid=(7,),
        in_specs=in_specs,
        out_specs=pl.BlockSpec((npad, _CP), lambda k: (k, 0)),
        out_shape=jax.ShapeDtypeStruct((7 * npad, _CP), jnp.float32),
        scratch_shapes=[pltpu.VMEM((npad, _CP), jnp.float32)],
        compiler_params=pltpu.CompilerParams(
            dimension_semantics=("arbitrary",)),
    )


def _make_tc_finalize(wt, n):
    def body(stats_ref, g_ref, be_ref, o_ref):
        scale, shift = _finalize_stats(stats_ref[...], g_ref[...],
                                       be_ref[...], n)
        o_ref[...] = jnp.concatenate([scale[None, :], shift[None, :]],
                                     axis=0)

    return pl.pallas_call(
        body, out_shape=jax.ShapeDtypeStruct((2, _CP), jnp.float32))


def _onehot_pool(noTp, npad_c, npad_f):
    cols = lax.broadcasted_iota(jnp.int32, (npad_c, npad_f), 1)
    m = None
    for kk in range(7):
        p = (noTp[kk][:, None] == cols).astype(jnp.float32)
        m = p if m is None else m + p
    return m * (1.0 / 7.0)


def _make_tc_mega(n_f, npad_f, c_f, n_c, npad_c, specs, last, raw_in=False):
    (ci1, co1), (ci2, co2) = specs

    def body(actf_ref, *refs):
        if raw_in:
            (stats_ref, gf_ref, bef_ref, noTp_ref, noT_ref, w1_ref,
             g1_ref, be1_ref, w2_ref, g2_ref, be2_ref, o_ref) = refs
            scale, shift = _finalize_stats(stats_ref[...], gf_ref[...],
                                           bef_ref[...], n_f)
            a = actf_ref[...] * scale[None, :] + shift[None, :]
            a = jnp.where(a >= 0, a, a * 0.1)
            rid = lax.broadcasted_iota(jnp.int32, (npad_f, a.shape[1]), 0)
            actf = jnp.where(rid < n_f, a, 0.0)
        else:
            (noTp_ref, noT_ref, w1_ref, g1_ref, be1_ref, w2_ref, g2_ref,
             be2_ref, o_ref) = refs
            actf = actf_ref[...]
        M = _onehot_pool(noTp_ref[...], npad_c, npad_f)
        x = jnp.dot(M, actf, preferred_element_type=jnp.float32)
        x = x[:, :ci1]
        cols = lax.broadcasted_iota(jnp.int32, (npad_c, npad_c), 1)
        noT = noT_ref[...]
        Pk = [(noT[kk][:, None] == cols).astype(jnp.float32)
              for kk in range(7)]
        for (ci, co, w_ref, g_ref, be_ref) in (
                (ci1, co1, w1_ref, g1_ref, be1_ref),
                (ci2, co2, w2_ref, g2_ref, be2_ref)):
            mat = jnp.concatenate(
                [jnp.dot(p, x, preferred_element_type=jnp.float32)
                 for p in Pk], axis=1)
            w = w_ref[...]
            if w.dtype == jnp.bfloat16:
                mat = mat.astype(jnp.bfloat16)
            out = jnp.dot(mat, w, preferred_element_type=jnp.float32)
            x = _bn_act(out, g_ref[...], be_ref[...], n_c, npad_c, co)
        if last:
            o_ref[...] = jnp.sum(x, axis=0, keepdims=True) * (1.0 / n_c)
        else:
            o_ref[...] = x

    out_shape = jax.ShapeDtypeStruct(
        (1, co2) if last else (npad_c, co2), jnp.float32)
    return pl.pallas_call(body, out_shape=out_shape)


def _pad2(a, rows, cols):
    return jnp.pad(a, ((0, rows - a.shape[0]), (0, cols - a.shape[1])))


def kernel(x, no0, no1, no2, no3, no4, no5,
           W0, b0, g0, be0, W1, b1, g1, be1, W2, b2, g2, be2,
           W3, b3, g3, be3, W4, b4, g4, be4, W5, b5, g5, be5,
           W6, b6, g6, be6, W7, b7, g7, be7, W8, b8, g8, be8,
           W9, b9, g9, be9, W10, b10, g10, be10, W11, b11, g11, be11,
           W12, b12, g12, be12):
    nos = [no0, no1, no2, no3, no4, no5]
    Ws = [W0, W1, W2, W3, W4, W5, W6, W7, W8, W9, W10, W11, W12]
    gs = [g0, g1, g2, g3, g4, g5, g6, g7, g8, g9, g10, g11, g12]
    bes = [be0, be1, be2, be3, be4, be5, be6, be7, be8, be9, be10, be11,
           be12]

    geoms = [_geom(n) for n in _LEVELS[:2]]

    sig32 = [2 * p for p in range(16)] + [2 * p + 1 for p in range(16)]
    sig = jnp.asarray([g * 32 + p for g in range(_CP // 32) for p in sig32],
                      dtype=jnp.int32)

    noflats = []
    for l in range(2):
        n = _LEVELS[l]
        _, _, npad = geoms[l]
        noflats.append(jnp.pad(nos[l], (0, 7 * (npad - n)),
                               constant_values=n))
    n_c1 = _LEVELS[1]
    ch1, wt1, npad1 = geoms[1]
    noflat_p1 = jnp.pad(nos[0][:n_c1 * 7], (0, 7 * (npad1 - n_c1)))

    W3d = []
    g2s = []
    be2s = []
    for j, (ci, co, lv) in enumerate(_SPECS):
        if j <= 4:
            w = Ws[j].reshape(7, ci, co)
            w = jnp.pad(w, ((0, 0), (0, _CP - ci), (0, _CP - co)))
            W3d.append(w)
        else:
            W3d.append(Ws[j])
        c = max(co, _CP)
        g2 = jnp.pad(gs[j], (0, c - co)).reshape(1, c)
        be2 = jnp.pad(bes[j], (0, c - co)).reshape(1, c)
        g2s.append(g2)
        be2s.append(be2)

    act = x
    state_raw = None
    out = stats = None
    for j in range(5):
        ci, co, lv = _SPECS[j]
        n = _LEVELS[lv]
        ch, wt, npad = geoms[lv]
        if state_raw is None:
            Y = _make_tc_mm(npad, n, raw=False)(act, W3d[j])
        else:
            out_p, stats_p = state_raw
            Y = _make_tc_mm(npad, n, raw=True, wt=wt)(
                out_p, stats_p, g2s[j - 1], be2s[j - 1], W3d[j])
        out, stats = _make_sc_conv(ch, wt, npad)(Y, noflats[lv])
        state_raw = (out, stats)

        if j == 2:
            _, wt0, _ = geoms[0]
            act = _make_sc_pool(ch1, wt1, npad1, wt0, _LEVELS[0])(
                out, noflat_p1, stats, g2s[2], be2s[2])
            state_raw = None

    npads = {1: npad1, 2: 672, 3: 168, 4: 48, 5: 16}
    for l in (2, 3, 4, 5):
        n_f, n_c = _LEVELS[l - 1], _LEVELS[l]
        npad_f, npad_c = npads[l - 1], npads[l]
        j1, j2 = 2 * l + 1, 2 * l + 2
        ci1, co1, _ = _SPECS[j1]
        ci2, co2, _ = _SPECS[j2]
        tp = nos[l - 1][:n_c * 7].reshape(n_c, 7).T
        tp = jnp.pad(tp, ((0, 0), (0, npad_c - n_c)), constant_values=0)
        t = nos[l].reshape(n_c, 7).T
        t = jnp.pad(t, ((0, 0), (0, npad_c - n_c)), constant_values=0)
        w1m, w2m = Ws[j1], Ws[j2]
        mega = _make_tc_mega(n_f, npad_f, max(ci1, _CP), n_c, npad_c,
                             [(ci1, co1), (ci2, co2)], last=(l == 5),
                             raw_in=(l == 2))
        if l == 2:
            act = mega(out, stats, g2s[4], be2s[4], tp, t,
                       w1m, g2s[j1], be2s[j1],
                       w2m, g2s[j2], be2s[j2])
        else:
            act = mega(act, tp, t, w1m, g2s[j1], be2s[j1],
                       w2m, g2s[j2], be2s[j2])
    return act
